# baseline (speedup 1.0000x reference)
import functools

import jax
import jax.numpy as jnp
from jax import lax
from jax.experimental import pallas as pl
from jax.experimental.pallas import tpu as pltpu
from jax.experimental.pallas import tpu_sc as plsc

LANES = 16
NC = 2
NS = 16
NW = NC * NS
CHUNK = 16384
UNROLL = 4


def _sc_body(nchunks, n_in, n_out, bpw,
             rc_h, w_h, x_h, bias_h, out_h,
             x_v, acc_v, rc_b0, w_b0, rc_b1, w_b1,
             sem_x, sem_a, sem_b):
  cid = lax.axis_index("c")
  sid = lax.axis_index("s")
  wid = sid * NC + cid

  cp_x = [pltpu.async_copy(x_h.at[pl.ds(wid * n_in, n_in)], x_v, sem_x)]

  def chunk_off(g):
    return lax.rem(wid + g, nchunks) * CHUNK

  sems = (sem_a, sem_b)
  bufs = ((rc_b0, w_b0), (rc_b1, w_b1))
  off0 = chunk_off(0)
  pending = [
      pltpu.async_copy(rc_h.at[pl.ds(off0, CHUNK)], rc_b0, sem_a),
      pltpu.async_copy(w_h.at[pl.ds(off0, CHUNK)], w_b0, sem_a),
  ]

  for b in range(bpw):
    pltpu.sync_copy(bias_h, acc_v[b])
  for cp in cp_x:
    cp.wait()

  for g in range(nchunks):
    slot = g % 2
    for cp in pending:
      cp.wait()
    if g + 1 < nchunks:
      nxt = slot ^ 1
      off = chunk_off(g + 1)
      sem = sems[nxt]
      pending = [
          pltpu.async_copy(rc_h.at[pl.ds(off, CHUNK)], bufs[nxt][0], sem),
          pltpu.async_copy(w_h.at[pl.ds(off, CHUNK)], bufs[nxt][1], sem),
      ]
    else:
      pending = []

    rcb, wb = bufs[slot]

    @plsc.parallel_loop(0, CHUNK // (2 * LANES), unroll=UNROLL)
    def _(i):
      o32 = pl.multiple_of(i * 2 * LANES, 2 * LANES)
      wpair = plsc.unpack(
          wb[pl.ds(o32, 2 * LANES)], format=plsc.PackFormat.INTERLEAVED)
      for half in range(2):
        o = o32 + half * LANES
        rcv = rcb[pl.ds(o, LANES)]
        wv = wpair[half]
        rv = lax.bitwise_and(rcv, jnp.int32(0xFFFF))
        cv = lax.shift_right_logical(rcv, jnp.int32(16))
        xp = plsc.load_gather(x_v, [rv])
        x0 = plsc.bitcast(lax.shift_left(xp, jnp.int32(16)), jnp.float32)
        x1 = plsc.bitcast(
            lax.bitwise_and(xp, jnp.int32(-65536)), jnp.float32)
        plsc.addupdate_scatter(acc_v[0], [cv], wv * x0)
        plsc.addupdate_scatter(acc_v[1], [cv], wv * x1)

  for b in range(bpw):
    pltpu.sync_copy(acc_v[b], out_h.at[pl.ds((wid * bpw + b) * n_out, n_out)])


def kernel(x, weights, bias, connections):
  batch, n_in = x.shape
  n_out = bias.shape[0]
  nnz = weights.shape[0]
  bpw = batch // NW

  nchunks = -(-nnz // CHUNK)
  pad = nchunks * CHUNK - nnz

  rc = lax.shift_left(connections[:, 1], 16) | connections[:, 0]
  if pad:
    rc = jnp.concatenate([rc, jnp.zeros((pad,), jnp.int32)])
    weights = jnp.concatenate([weights, jnp.zeros((pad,), jnp.float32)])
  wp = weights.reshape(-1, 2, LANES).swapaxes(1, 2).reshape(-1).astype(
      jnp.bfloat16)

  xb = lax.bitcast_convert_type(x.astype(jnp.bfloat16), jnp.uint16)
  xp = lax.bitcast_convert_type(
      (xb[1::2].astype(jnp.uint32) << 16) | xb[0::2].astype(jnp.uint32),
      jnp.int32)

  mesh = plsc.VectorSubcoreMesh(
      core_axis_name="c", subcore_axis_name="s", num_cores=NC,
      num_subcores=NS)
  body = functools.partial(_sc_body, nchunks, n_in, n_out, bpw)
  out_flat = pl.kernel(
      body,
      out_type=jax.ShapeDtypeStruct((batch * n_out,), jnp.float32),
      mesh=mesh,
      compiler_params=pltpu.CompilerParams(needs_layout_passes=False),
      scratch_types=[
          pltpu.VMEM((n_in,), jnp.int32),
          [pltpu.VMEM((n_out,), jnp.float32) for _ in range(bpw)],
          pltpu.VMEM((CHUNK,), jnp.int32),
          pltpu.VMEM((CHUNK,), jnp.bfloat16),
          pltpu.VMEM((CHUNK,), jnp.int32),
          pltpu.VMEM((CHUNK,), jnp.bfloat16),
          pltpu.SemaphoreType.DMA,
          pltpu.SemaphoreType.DMA,
          pltpu.SemaphoreType.DMA,
      ],
  )(rc, wp, xp.reshape(-1), bias.reshape(-1))
  return out_flat.reshape(batch, n_out)


# speedup vs baseline: 1.5014x; 1.5014x over previous
import functools

import jax
import jax.numpy as jnp
from jax import lax
from jax.experimental import pallas as pl
from jax.experimental.pallas import tpu as pltpu
from jax.experimental.pallas import tpu_sc as plsc

LANES = 16
NC = 2
NS = 16
NW = NC * NS
CHUNK = 8192
UNROLL = 4


def _sc_body(nchunks, n_in, n_out, bpw,
             rc_h, w_h, x_h, bias_h, out_h,
             x_v, acc_v, rc_b0, w_b0, rc_b1, w_b1,
             sem_x, sem_a, sem_b):
  cid = lax.axis_index("c")
  sid = lax.axis_index("s")
  wid = sid * NC + cid

  cp_x = [pltpu.async_copy(x_h.at[pl.ds(wid * n_in, n_in)], x_v, sem_x)]

  def start_chunk(g, buf, sem):
    ci = lax.rem(wid + g, nchunks)
    return [
        pltpu.async_copy(
            rc_h.at[pl.ds(ci * CHUNK, CHUNK)], buf[0], sem),
        pltpu.async_copy(
            w_h.at[pl.ds(ci * (CHUNK // 2), CHUNK // 2)], buf[1], sem),
    ]

  sems = (sem_a, sem_b)
  bufs = ((rc_b0, w_b0), (rc_b1, w_b1))
  pending = start_chunk(0, bufs[0], sem_a)

  for b in range(bpw):
    pltpu.sync_copy(bias_h, acc_v[b])
  for cp in cp_x:
    cp.wait()

  for g in range(nchunks):
    slot = g % 2
    for cp in pending:
      cp.wait()
    if g + 1 < nchunks:
      nxt = slot ^ 1
      pending = start_chunk(g + 1, bufs[nxt], sems[nxt])
    else:
      pending = []

    rcb, wb = bufs[slot]

    def expand_lo(v):
      return plsc.bitcast(lax.shift_left(v, jnp.int32(16)), jnp.float32)

    def expand_hi(v):
      return plsc.bitcast(
          lax.bitwise_and(v, jnp.int32(-65536)), jnp.float32)

    @plsc.parallel_loop(0, CHUNK // (2 * LANES), unroll=UNROLL)
    def _(i):
      o = pl.multiple_of(i * 2 * LANES, 2 * LANES)
      wp = wb[pl.ds(pl.multiple_of(i * LANES, LANES), LANES)]
      for half, wv in ((0, expand_lo(wp)), (1, expand_hi(wp))):
        rcv = rcb[pl.ds(o + half * LANES, LANES)]
        rv = lax.bitwise_and(rcv, jnp.int32(0xFFFF))
        cv = lax.shift_right_logical(rcv, jnp.int32(16))
        xp = plsc.load_gather(x_v, [rv])
        plsc.addupdate_scatter(acc_v[0], [cv], wv * expand_lo(xp))
        plsc.addupdate_scatter(acc_v[1], [cv], wv * expand_hi(xp))

  for b in range(bpw):
    pltpu.sync_copy(acc_v[b], out_h.at[pl.ds((wid * bpw + b) * n_out, n_out)])


def kernel(x, weights, bias, connections):
  batch, n_in = x.shape
  n_out = bias.shape[0]
  nnz = weights.shape[0]
  bpw = batch // NW

  nchunks = -(-nnz // CHUNK)
  pad = nchunks * CHUNK - nnz

  rc = lax.shift_left(connections[:, 1], 16) | connections[:, 0]
  if pad:
    rc = jnp.concatenate([rc, jnp.zeros((pad,), jnp.int32)])
    weights = jnp.concatenate([weights, jnp.zeros((pad,), jnp.float32)])
  w16 = lax.bitcast_convert_type(
      weights.astype(jnp.bfloat16), jnp.uint16).reshape(-1, 2, LANES)
  wpk = lax.bitcast_convert_type(
      (w16[:, 1, :].astype(jnp.uint32) << 16) | w16[:, 0, :].astype(
          jnp.uint32), jnp.int32).reshape(-1)

  xb = lax.bitcast_convert_type(x.astype(jnp.bfloat16), jnp.uint16)
  xp = lax.bitcast_convert_type(
      (xb[1::2].astype(jnp.uint32) << 16) | xb[0::2].astype(jnp.uint32),
      jnp.int32)

  mesh = plsc.VectorSubcoreMesh(
      core_axis_name="c", subcore_axis_name="s", num_cores=NC,
      num_subcores=NS)
  body = functools.partial(_sc_body, nchunks, n_in, n_out, bpw)
  out_flat = pl.kernel(
      body,
      out_type=jax.ShapeDtypeStruct((batch * n_out,), jnp.float32),
      mesh=mesh,
      compiler_params=pltpu.CompilerParams(needs_layout_passes=False),
      scratch_types=[
          pltpu.VMEM((n_in,), jnp.int32),
          [pltpu.VMEM((n_out,), jnp.float32) for _ in range(bpw)],
          pltpu.VMEM((CHUNK,), jnp.int32),
          pltpu.VMEM((CHUNK // 2,), jnp.int32),
          pltpu.VMEM((CHUNK,), jnp.int32),
          pltpu.VMEM((CHUNK // 2,), jnp.int32),
          pltpu.SemaphoreType.DMA,
          pltpu.SemaphoreType.DMA,
          pltpu.SemaphoreType.DMA,
      ],
  )(rc, wpk, xp.reshape(-1), bias.reshape(-1))
  return out_flat.reshape(batch, n_out)


# speedup vs baseline: 1.8305x; 1.2192x over previous
import functools

import jax
import jax.numpy as jnp
from jax import lax
from jax.experimental import pallas as pl
from jax.experimental.pallas import tpu as pltpu
from jax.experimental.pallas import tpu_sc as plsc

LANES = 16
NC = 2
NS = 16
NW = NC * NS
BPW = 4
CHUNK = 4096
STG = 4096
UNROLL = 8


def _expand_lo(v):
  return plsc.bitcast(lax.shift_left(v, jnp.int32(16)), jnp.float32)


def _expand_hi(v):
  return plsc.bitcast(lax.bitwise_and(v, jnp.int32(-65536)), jnp.float32)


def _sc_body(nch2, n_in, n_out,
             rc_h, w_h, x_h, bias2_h, out_h,
             x_v, acc_v, rc_b0, w_b0, rc_b1, w_b1, stage, shared,
             sem_x, sem_a, sem_b):
  cid = lax.axis_index("c")
  sid = lax.axis_index("s")
  half = sid // (NS // 2)
  mate = sid % (NS // 2)
  group = cid * (NS // 2) + mate
  wid = sid * NC + cid

  cp_x = [
      pltpu.async_copy(
          x_h.at[pl.ds((group * 2 + j) * n_in, n_in)], x_v[j], sem_x)
      for j in range(2)
  ]

  def start_chunk(g, buf, sem):
    ci = half * nch2 + lax.rem(wid + g, nch2)
    return [
        pltpu.async_copy(rc_h.at[pl.ds(ci * CHUNK, CHUNK)], buf[0], sem),
        pltpu.async_copy(w_h.at[pl.ds(ci * CHUNK, CHUNK)], buf[1], sem),
    ]

  sems = (sem_a, sem_b)
  bufs = ((rc_b0, w_b0), (rc_b1, w_b1))
  pending = start_chunk(0, bufs[0], sem_a)

  for b in range(BPW):
    pltpu.sync_copy(bias2_h.at[pl.ds(half * n_out, n_out)], acc_v[b])
  for cp in cp_x:
    cp.wait()

  for g in range(nch2):
    slot = g % 2
    for cp in pending:
      cp.wait()
    if g + 1 < nch2:
      nxt = slot ^ 1
      pending = start_chunk(g + 1, bufs[nxt], sems[nxt])
    else:
      pending = []

    rcb, wb = bufs[slot]

    @plsc.parallel_loop(0, CHUNK // LANES, unroll=UNROLL)
    def _(i):
      o = pl.multiple_of(i * LANES, LANES)
      rcv = rcb[pl.ds(o, LANES)]
      wv = wb[pl.ds(o, LANES)]
      rv = lax.bitwise_and(rcv, jnp.int32(0xFFFF))
      cv = lax.shift_right_logical(rcv, jnp.int32(16))
      xp0 = plsc.load_gather(x_v[0], [rv])
      plsc.addupdate_scatter(acc_v[0], [cv], wv * _expand_lo(xp0))
      plsc.addupdate_scatter(acc_v[1], [cv], wv * _expand_hi(xp0))
      xp1 = plsc.load_gather(x_v[1], [rv])
      plsc.addupdate_scatter(acc_v[2], [cv], wv * _expand_lo(xp1))
      plsc.addupdate_scatter(acc_v[3], [cv], wv * _expand_hi(xp1))

  for b in range(BPW):
    @pl.when(half == 1)
    def _():
      pltpu.sync_copy(acc_v[b], shared.at[mate])
    plsc.subcore_barrier()

    @pl.when(half == 0)
    def _():
      for k in range(n_out // STG):
        pltpu.sync_copy(shared.at[mate, pl.ds(k * STG, STG)], stage)

        @plsc.parallel_loop(0, STG // LANES, unroll=UNROLL)
        def _(i):
          s = pl.multiple_of(i * LANES, LANES)
          o = s + k * STG
          acc_v[b][pl.ds(o, LANES)] = (
              acc_v[b][pl.ds(o, LANES)] + stage[pl.ds(s, LANES)])
    plsc.subcore_barrier()

  @pl.when(half == 0)
  def _():
    for b in range(BPW):
      pltpu.sync_copy(
          acc_v[b], out_h.at[pl.ds((group * BPW + b) * n_out, n_out)])


def kernel(x, weights, bias, connections):
  batch, n_in = x.shape
  n_out = bias.shape[0]
  nnz = weights.shape[0]

  nch2 = -(-nnz // (2 * CHUNK))
  pad = nch2 * 2 * CHUNK - nnz

  rc = lax.shift_left(connections[:, 1], 16) | connections[:, 0]
  if pad:
    rc = jnp.concatenate([rc, jnp.zeros((pad,), jnp.int32)])
    weights = jnp.concatenate([weights, jnp.zeros((pad,), jnp.float32)])

  xb = lax.bitcast_convert_type(x.astype(jnp.bfloat16), jnp.uint16)
  xp = lax.bitcast_convert_type(
      (xb[1::2].astype(jnp.uint32) << 16) | xb[0::2].astype(jnp.uint32),
      jnp.int32)

  bias2 = jnp.concatenate(
      [bias.reshape(-1), jnp.zeros((n_out,), jnp.float32)])

  mesh = plsc.VectorSubcoreMesh(
      core_axis_name="c", subcore_axis_name="s", num_cores=NC,
      num_subcores=NS)
  body = functools.partial(_sc_body, nch2, n_in, n_out)
  out_flat = pl.kernel(
      body,
      out_type=jax.ShapeDtypeStruct((batch * n_out,), jnp.float32),
      mesh=mesh,
      compiler_params=pltpu.CompilerParams(needs_layout_passes=False),
      scratch_types=[
          [pltpu.VMEM((n_in,), jnp.int32) for _ in range(2)],
          [pltpu.VMEM((n_out,), jnp.float32) for _ in range(BPW)],
          pltpu.VMEM((CHUNK,), jnp.int32),
          pltpu.VMEM((CHUNK,), jnp.float32),
          pltpu.VMEM((CHUNK,), jnp.int32),
          pltpu.VMEM((CHUNK,), jnp.float32),
          pltpu.VMEM((STG,), jnp.float32),
          pltpu.VMEM_SHARED((NS // 2, n_out), jnp.float32),
          pltpu.SemaphoreType.DMA,
          pltpu.SemaphoreType.DMA,
          pltpu.SemaphoreType.DMA,
      ],
  )(rc, weights, xp.reshape(-1), bias2)
  return out_flat.reshape(batch, n_out)
